# Initial kernel scaffold; baseline (speedup 1.0000x reference)
import jax
import jax.numpy as jnp
from jax import lax
from jax.experimental import pallas as pl
from jax.experimental.pallas import tpu as pltpu
from jax.experimental.pallas import tpu_sc as plsc

VOCAB = 100000
MAXPOS = 512
EMB = 128
B = 1024
S = 512
EPS = 1e-3

NC = 2
NS = 16
NW = NC * NS
L = 16
NV = EMB // L

TOK = B * S
TPW = TOK // NW
CHUNK = 128
NCH = TPW // CHUNK

_GDN = lax.GatherDimensionNumbers(
    offset_dims=(), collapsed_slice_dims=(0,), start_index_map=(0,))


def _shuf(v, idx):
    return lax.gather(v, idx[:, None], _GDN, (1,),
                      mode=lax.GatherScatterMode.PROMISE_IN_BOUNDS)


def _body(ids_ref, tt_ref, table_ref, pos_ref, type_ref, gamma_ref, beta_ref,
          out_ref, pos_v, aux_v, idx_v, tt_i, ttf_v, rows_v, sem):
    c = lax.axis_index("c")
    s = lax.axis_index("s")
    wid = s * NC + c
    base = wid * TPW

    pltpu.sync_copy(pos_ref, pos_v)
    pltpu.sync_copy(gamma_ref, aux_v.at[0])
    pltpu.sync_copy(beta_ref, aux_v.at[1])
    pltpu.sync_copy(type_ref, aux_v.at[pl.ds(2, 2)])

    gam = [aux_v[0, pl.ds(k * L, L)] for k in range(NV)]
    bet = [aux_v[1, pl.ds(k * L, L)] for k in range(NV)]
    ty0 = [aux_v[2, pl.ds(k * L, L)] for k in range(NV)]
    dty = [aux_v[3, pl.ds(k * L, L)] - ty0[k] for k in range(NV)]
    lanes = lax.iota(jnp.int32, L)
    bfly = [lax.bitwise_xor(lanes, m) for m in (1, 2, 4, 8)]

    def allsum(v):
        for ix in bfly:
            v = v + _shuf(v, ix)
        return v

    def chunk_body(ch, carry):
        gbase = base + ch * CHUNK
        pltpu.sync_copy(ids_ref.at[pl.ds(gbase, CHUNK)], idx_v)
        pltpu.sync_copy(tt_ref.at[pl.ds(gbase, CHUNK)], tt_i)
        gath = pltpu.async_copy(table_ref.at[idx_v], rows_v, sem)
        for k in range(NV):
            ttf_v[pl.ds(k * L, L)] = tt_i[pl.ds(k * L, L)].astype(jnp.float32)
        smod = lax.rem(ch * CHUNK, S)
        gath.wait()

        def group(g, carry2):
            jbase = g * L
            ttg = ttf_v[pl.ds(jbase, L)]
            for jj in range(L):
                j = jbase + jj
                ttb = _shuf(ttg, lax.full((L,), jj, jnp.int32))
                srow = smod + j
                xs = []
                for k in range(NV):
                    x = rows_v[j, pl.ds(k * L, L)]
                    x = x + pos_v[srow, pl.ds(k * L, L)]
                    x = x + ty0[k] + ttb * dty[k]
                    xs.append(x)
                tot = (((xs[0] + xs[1]) + (xs[2] + xs[3]))
                       + ((xs[4] + xs[5]) + (xs[6] + xs[7])))
                sq = (((xs[0] * xs[0] + xs[1] * xs[1])
                       + (xs[2] * xs[2] + xs[3] * xs[3]))
                      + ((xs[4] * xs[4] + xs[5] * xs[5])
                         + (xs[6] * xs[6] + xs[7] * xs[7])))
                mean = allsum(tot) * (1.0 / EMB)
                ex2 = allsum(sq) * (1.0 / EMB)
                vv = ex2 - mean * mean + EPS
                bi = plsc.bitcast(vv, jnp.int32)
                bi = 0x5F3759DF - lax.shift_right_logical(bi, 1)
                y = plsc.bitcast(bi, jnp.float32)
                for _ in range(3):
                    y = y * (1.5 - 0.5 * vv * y * y)
                for k in range(NV):
                    rows_v[j, pl.ds(k * L, L)] = (
                        (xs[k] - mean) * y * gam[k] + bet[k])
            return carry2

        lax.fori_loop(0, CHUNK // L, group, 0)
        pltpu.sync_copy(rows_v, out_ref.at[pl.ds(gbase, CHUNK)])
        return carry

    lax.fori_loop(0, NCH, chunk_body, 0)


@jax.jit
def _run(ids_flat, tt_flat, table, pos, typ, gamma, beta):
    kern = pl.kernel(
        _body,
        out_type=jax.ShapeDtypeStruct((TOK, EMB), jnp.float32),
        mesh=plsc.VectorSubcoreMesh(core_axis_name="c", subcore_axis_name="s"),
        scratch_types=[
            pltpu.VMEM((MAXPOS, EMB), jnp.float32),
            pltpu.VMEM((4, EMB), jnp.float32),
            pltpu.VMEM((CHUNK,), jnp.int32),
            pltpu.VMEM((CHUNK,), jnp.int32),
            pltpu.VMEM((CHUNK,), jnp.float32),
            pltpu.VMEM((CHUNK, EMB), jnp.float32),
            pltpu.SemaphoreType.DMA,
        ],
    )
    return kern(ids_flat, tt_flat, table, pos, typ, gamma, beta)


def kernel(input_ids, token_type_ids, token_embedding, position_table,
           type_table, gamma, beta):
    ids_flat = input_ids.reshape(-1).astype(jnp.int32)
    tt_flat = token_type_ids.reshape(-1).astype(jnp.int32)
    out = _run(ids_flat, tt_flat, token_embedding, position_table,
               type_table, gamma, beta)
    return (out.reshape(B, S, EMB), token_embedding)


# speedup vs baseline: 2.9344x; 2.9344x over previous
import jax
import jax.numpy as jnp
from jax import lax
from jax.experimental import pallas as pl
from jax.experimental.pallas import tpu as pltpu
from jax.experimental.pallas import tpu_sc as plsc

VOCAB = 100000
MAXPOS = 512
EMB = 128
B = 1024
S = 512
EPS = 1e-3

NC = 2
NS = 16
NW = NC * NS
L = 16
NV = EMB // L

TOK = B * S
TPW = TOK // NW
CHUNK = 128
NCH = TPW // CHUNK

_GDN = lax.GatherDimensionNumbers(
    offset_dims=(), collapsed_slice_dims=(0,), start_index_map=(0,))


def _shuf(v, idx):
    return lax.gather(v, idx[:, None], _GDN, (1,),
                      mode=lax.GatherScatterMode.PROMISE_IN_BOUNDS)


def _body(ids_ref, tt_ref, table_ref, pos_ref, type_ref, gamma_ref, beta_ref,
          out_ref, pos_v, aux_v, idx_v, tt_i, ttf_v, rows_v, sem):
    c = lax.axis_index("c")
    s = lax.axis_index("s")
    wid = s * NC + c
    base = wid * TPW

    pltpu.sync_copy(pos_ref, pos_v)
    pltpu.sync_copy(gamma_ref, aux_v.at[0])
    pltpu.sync_copy(beta_ref, aux_v.at[1])
    pltpu.sync_copy(type_ref, aux_v.at[pl.ds(2, 2)])

    gam = [aux_v[0, pl.ds(k * L, L)] for k in range(NV)]
    bet = [aux_v[1, pl.ds(k * L, L)] for k in range(NV)]
    ty0 = [aux_v[2, pl.ds(k * L, L)] for k in range(NV)]
    dty = [aux_v[3, pl.ds(k * L, L)] - ty0[k] for k in range(NV)]
    lanes = lax.iota(jnp.int32, L)
    bfly = [lax.bitwise_xor(lanes, m) for m in (1, 2, 4, 8)]

    def allsum(v):
        for ix in bfly:
            v = v + _shuf(v, ix)
        return v

    def chunk_body(ch, carry):
        gbase = base + ch * CHUNK
        pltpu.sync_copy(ids_ref.at[pl.ds(gbase, CHUNK)], idx_v)
        pltpu.sync_copy(tt_ref.at[pl.ds(gbase, CHUNK)], tt_i)
        gath = pltpu.async_copy(table_ref.at[idx_v], rows_v, sem)
        for k in range(NV):
            ttf_v[pl.ds(k * L, L)] = tt_i[pl.ds(k * L, L)].astype(jnp.float32)
        smod = lax.rem(ch * CHUNK, S)
        gath.wait()

        def group(g, carry2):
            jbase = g * L
            ttg = ttf_v[pl.ds(jbase, L)]
            for jj in range(L):
                j = jbase + jj
                ttb = _shuf(ttg, lax.full((L,), jj, jnp.int32))
                srow = smod + j
                xs = []
                for k in range(NV):
                    x = rows_v[j, pl.ds(k * L, L)]
                    x = x + pos_v[srow, pl.ds(k * L, L)]
                    x = x + ty0[k] + ttb * dty[k]
                    xs.append(x)
                tot = (((xs[0] + xs[1]) + (xs[2] + xs[3]))
                       + ((xs[4] + xs[5]) + (xs[6] + xs[7])))
                sq = (((xs[0] * xs[0] + xs[1] * xs[1])
                       + (xs[2] * xs[2] + xs[3] * xs[3]))
                      + ((xs[4] * xs[4] + xs[5] * xs[5])
                         + (xs[6] * xs[6] + xs[7] * xs[7])))
                mean = allsum(tot) * (1.0 / EMB)
                ex2 = allsum(sq) * (1.0 / EMB)
                vv = ex2 - mean * mean + EPS
                bi = lax.bitcast_convert_type(vv, jnp.int32)
                bi = 0x5F3759DF - lax.shift_right_logical(bi, 1)
                y = lax.bitcast_convert_type(bi, jnp.float32)
                for _ in range(3):
                    y = y * (1.5 - 0.5 * vv * y * y)
                for k in range(NV):
                    rows_v[j, pl.ds(k * L, L)] = (
                        (xs[k] - mean) * y * gam[k] + bet[k])
            return carry2

        lax.fori_loop(0, CHUNK // L, group, 0)
        pltpu.sync_copy(rows_v, out_ref.at[pl.ds(gbase, CHUNK)])
        return carry

    lax.fori_loop(0, NCH, chunk_body, 0)


@jax.jit
def _run(ids_flat, tt_flat, table, pos, typ, gamma, beta):
    kern = pl.kernel(
        _body,
        out_type=jax.ShapeDtypeStruct((TOK, EMB), jnp.float32),
        mesh=plsc.VectorSubcoreMesh(core_axis_name="c", subcore_axis_name="s"),
        scratch_types=[
            pltpu.VMEM((MAXPOS, EMB), jnp.float32),
            pltpu.VMEM((4, EMB), jnp.float32),
            pltpu.VMEM((CHUNK,), jnp.int32),
            pltpu.VMEM((CHUNK,), jnp.int32),
            pltpu.VMEM((CHUNK,), jnp.float32),
            pltpu.VMEM((CHUNK, EMB), jnp.float32),
            pltpu.SemaphoreType.DMA,
        ],
    )
    return kern(ids_flat, tt_flat, table, pos, typ, gamma, beta)


def kernel(input_ids, token_type_ids, token_embedding, position_table,
           type_table, gamma, beta):
    ids_flat = input_ids.reshape(-1).astype(jnp.int32)
    tt_flat = token_type_ids.reshape(-1).astype(jnp.int32)
    out = _run(ids_flat, tt_flat, token_embedding, position_table,
               type_table, gamma, beta)
    return (out.reshape(B, S, EMB), token_embedding)
